# Initial kernel scaffold; baseline (speedup 1.0000x reference)
import jax
import jax.numpy as jnp
from jax.experimental import pallas as pl


def kernel(input, tar_group_weights, enc_weights, group_edge_ids, group_edge_weights, edge_ids, edge_weights, emb_w, emb_b, g_fc1_w, g_fc1_b, g_fc2_w, g_fc2_b, glob_fc1_w, glob_fc1_b, glob_fc2_w, glob_fc2_b):
    raise NotImplementedError("write your pallas kernel here")


# speedup vs baseline: 2.1048x; 2.1048x over previous
import functools

import jax
import jax.numpy as jnp
from jax import lax
from jax.experimental import pallas as pl
from jax.experimental.pallas import tpu as pltpu
from jax.experimental.pallas import tpu_sc as plsc

N = 10000
E = 160000
G = 64
EG = 2048
D = 256
H = 128
ACCW = 144

NC = 2
NS = 16
L = 16

EPT = E // NS
B = 80
NBLK = EPT // B
CHUNK = 25
NCHUNK = NBLK // CHUNK
RPT = N // NS

BN = 1000
NB = N // BN

_F32 = jnp.float32


def _a_body(inp_ref, embT_ref, embb_ref, tgw_ref, x_ref, gacc_ref):
    x = jnp.dot(inp_ref[...], embT_ref[...], preferred_element_type=_F32)
    x = x + embb_ref[...]
    x_ref[...] = x
    part = lax.dot_general(tgw_ref[...], x, (((0,), (0,)), ((), ())),
                           preferred_element_type=_F32)
    @pl.when(pl.program_id(0) == 0)
    def _():
        gacc_ref[...] = part
    @pl.when(pl.program_id(0) != 0)
    def _():
        gacc_ref[...] += part


def _call_a(inp, embT, embb, tgw):
    return pl.pallas_call(
        _a_body,
        grid=(NB,),
        in_specs=[
            pl.BlockSpec((BN, D), lambda i: (i, 0)),
            pl.BlockSpec((D, D), lambda i: (0, 0)),
            pl.BlockSpec((1, D), lambda i: (0, 0)),
            pl.BlockSpec((BN, G), lambda i: (i, 0)),
        ],
        out_specs=[
            pl.BlockSpec((BN, D), lambda i: (i, 0)),
            pl.BlockSpec((G, D), lambda i: (0, 0)),
        ],
        out_shape=[
            jax.ShapeDtypeStruct((N, D), _F32),
            jax.ShapeDtypeStruct((G, D), _F32),
        ],
    )(inp, embT, embb, tgw)


def _b_body(gx0_ref, grow_ref, gcol_ref, gew_ref, g1xT_ref, g1eT_ref,
            g1b_ref, g2xT_ref, g2aT_ref, g2b_ref, gx_ref):
    gx0 = gx0_ref[...]
    ids = lax.broadcasted_iota(jnp.int32, (EG, G), 1)
    oh_r = (grow_ref[...] == ids).astype(_F32)
    oh_c = (gcol_ref[...] == ids).astype(_F32)
    base = jnp.dot(gx0, g1xT_ref[...], preferred_element_type=_F32)
    m = jnp.dot(oh_r, base, preferred_element_type=_F32)
    m = m + jnp.dot(gew_ref[...], g1eT_ref[...], preferred_element_type=_F32)
    m = jnp.maximum(m + g1b_ref[...], 0.0)
    sums = lax.dot_general(oh_c, m, (((0,), (0,)), ((), ())),
                           preferred_element_type=_F32)
    cnt = jnp.sum(oh_c, axis=0, keepdims=True)
    agg = sums / jnp.maximum(cnt, 1.0).reshape(G, 1)
    h = jnp.dot(gx0, g2xT_ref[...], preferred_element_type=_F32)
    h = h + jnp.dot(agg, g2aT_ref[...], preferred_element_type=_F32)
    gx_ref[...] = jnp.maximum(h + g2b_ref[...], 0.0)


def _call_b(gx0, grow, gcol, gew, g1xT, g1eT, g1b, g2xT, g2aT, g2b):
    return pl.pallas_call(
        _b_body,
        out_shape=jax.ShapeDtypeStruct((G, D), _F32),
    )(gx0, grow, gcol, gew, g1xT, g1eT, g1b, g2xT, g2aT, g2b)


def _c_body(x_ref, tgw_ref, gx_ref, w1xT_ref, w1nT_ref, b1_ref,
            w2xT_ref, w2nT_ref, b2_ref, y2_ref, z_ref):
    x = x_ref[...]
    new_x = jnp.dot(tgw_ref[...], gx_ref[...], preferred_element_type=_F32)
    y = jnp.dot(x, w1xT_ref[...], preferred_element_type=_F32)
    y = y + jnp.dot(new_x, w1nT_ref[...], preferred_element_type=_F32)
    y = y + b1_ref[...]
    y2_ref[0] = y[:, :H]
    y2_ref[1] = y[:, H:]
    z = jnp.dot(x, w2xT_ref[...], preferred_element_type=_F32)
    z = z + jnp.dot(new_x, w2nT_ref[...], preferred_element_type=_F32)
    z_ref[...] = z + b2_ref[...]


def _call_c(x, tgw, gx, w1xT, w1nT, b1, w2xT, w2nT, b2):
    return pl.pallas_call(
        _c_body,
        grid=(NB,),
        in_specs=[
            pl.BlockSpec((BN, D), lambda i: (i, 0)),
            pl.BlockSpec((BN, G), lambda i: (i, 0)),
            pl.BlockSpec((G, D), lambda i: (0, 0)),
            pl.BlockSpec((D, D), lambda i: (0, 0)),
            pl.BlockSpec((D, D), lambda i: (0, 0)),
            pl.BlockSpec((1, D), lambda i: (0, 0)),
            pl.BlockSpec((D, D), lambda i: (0, 0)),
            pl.BlockSpec((D, D), lambda i: (0, 0)),
            pl.BlockSpec((1, D), lambda i: (0, 0)),
        ],
        out_specs=[
            pl.BlockSpec((2, BN, H), lambda i: (0, i, 0)),
            pl.BlockSpec((BN, D), lambda i: (i, 0)),
        ],
        out_shape=[
            jax.ShapeDtypeStruct((2, N, H), _F32),
            jax.ShapeDtypeStruct((N, D), _F32),
        ],
    )(x, tgw, gx, w1xT, w1nT, b1, w2xT, w2nT, b2)


def _edge_body(ytab, ice, w1b2, acc_out,
               ibuf, gbuf, obuf, w1b_v, acc_sh, sem):
    c = lax.axis_index("c")
    s = lax.axis_index("s")

    pltpu.sync_copy(w1b2.at[c], w1b_v)

    zero = jnp.zeros((L,), _F32)

    def _zrow(e, carry):
        for f in range(ACCW // L):
            obuf[e, pl.ds(L * f, L)] = zero
        return carry

    lax.fori_loop(0, B, _zrow, 0)
    nfull = RPT // B
    for k in range(nfull):
        pltpu.sync_copy(obuf, acc_sh.at[pl.ds(s * RPT + k * B, B)])
    rem = RPT - nfull * B
    if rem:
        pltpu.sync_copy(obuf.at[pl.ds(0, rem)],
                        acc_sh.at[pl.ds(s * RPT + nfull * B, rem)])
    plsc.subcore_barrier()

    w1bs = tuple(w1b_v[pl.ds(L * f, L)] for f in range(H // L))
    lane0 = jnp.where(lax.iota(jnp.int32, L) == 0, 1.0, 0.0).astype(_F32)

    def _chunk(ch, carry):
        pltpu.sync_copy(ice.at[c, s, ch], ibuf)

        def _block(jj, bcarry):
            pltpu.async_copy(ytab.at[ibuf.at[jj, 0]], gbuf, sem).wait()

            def _edge(e, ecarry):
                ew_vec = plsc.bitcast(
                    plsc.load_gather(
                        ibuf, [jnp.full((L,), jj, jnp.int32),
                               jnp.full((L,), 2, jnp.int32),
                               jnp.full((L,), e, jnp.int32)]), _F32)
                for f in range(H // L):
                    g = gbuf[e, pl.ds(L * f, L)]
                    obuf[e, pl.ds(L * f, L)] = jnp.maximum(
                        g + ew_vec * w1bs[f], 0.0)
                obuf[e, pl.ds(H, L)] = lane0
                return ecarry

            lax.fori_loop(0, B, _edge, 0)
            pltpu.sync_copy(obuf, acc_sh.at[ibuf.at[jj, 1]], add=True)
            return bcarry

        lax.fori_loop(0, CHUNK, _block, 0)
        return carry

    lax.fori_loop(0, NCHUNK, _chunk, 0)
    plsc.subcore_barrier()
    pltpu.sync_copy(acc_sh.at[pl.ds(s * RPT, RPT)],
                    acc_out.at[c, pl.ds(s * RPT, RPT)])


def _edge_call(ytab, ice, w1b2):
    mesh = plsc.VectorSubcoreMesh(core_axis_name="c", subcore_axis_name="s",
                                  num_cores=NC, num_subcores=NS)
    return pl.kernel(
        _edge_body,
        out_type=jax.ShapeDtypeStruct((NC, N, ACCW), _F32),
        mesh=mesh,
        scratch_types=[
            pltpu.VMEM((CHUNK, 3, B), jnp.int32),
            pltpu.VMEM((B, H), _F32),
            pltpu.VMEM((B, ACCW), _F32),
            pltpu.VMEM((H,), _F32),
            pltpu.VMEM_SHARED((N, ACCW), _F32),
            pltpu.SemaphoreType.DMA,
        ],
        compiler_params=pltpu.CompilerParams(use_tc_tiling_on_sc=False,
                                             needs_layout_passes=False),
    )(ytab, ice, w1b2)


def _e_body(z_ref, s0_ref, s1_ref, cnt_ref, w2aT_ref, w2bT_ref, out_ref):
    cnt = jnp.maximum(cnt_ref[...], 1.0)
    a0 = s0_ref[...] / cnt
    a1 = s1_ref[...] / cnt
    o = z_ref[...]
    o = o + jnp.dot(a0, w2aT_ref[...], preferred_element_type=_F32)
    o = o + jnp.dot(a1, w2bT_ref[...], preferred_element_type=_F32)
    out_ref[...] = jnp.maximum(o, 0.0)


def _call_e(z, s0, s1, cnt, w2aT, w2bT):
    return pl.pallas_call(
        _e_body,
        grid=(NB,),
        in_specs=[
            pl.BlockSpec((BN, D), lambda i: (i, 0)),
            pl.BlockSpec((BN, H), lambda i: (i, 0)),
            pl.BlockSpec((BN, H), lambda i: (i, 0)),
            pl.BlockSpec((BN, 1), lambda i: (i, 0)),
            pl.BlockSpec((H, D), lambda i: (0, 0)),
            pl.BlockSpec((H, D), lambda i: (0, 0)),
        ],
        out_specs=pl.BlockSpec((BN, D), lambda i: (i, 0)),
        out_shape=jax.ShapeDtypeStruct((N, D), _F32),
    )(z, s0, s1, cnt, w2aT, w2bT)


def kernel(input, tar_group_weights, enc_weights, group_edge_ids,
           group_edge_weights, edge_ids, edge_weights, emb_w, emb_b,
           g_fc1_w, g_fc1_b, g_fc2_w, g_fc2_b, glob_fc1_w, glob_fc1_b,
           glob_fc2_w, glob_fc2_b):
    f32 = _F32
    embT = emb_w.T
    embb = emb_b.reshape(1, D)
    g1xT = g_fc1_w[:, :D].T
    g1eT = g_fc1_w[:, D:].T
    g1b = g_fc1_b.reshape(1, D)
    g2xT = g_fc2_w[:, :D].T
    g2aT = g_fc2_w[:, D:].T
    g2b = g_fc2_b.reshape(1, D)
    w1xT = glob_fc1_w[:, :D].T
    w1nT = glob_fc1_w[:, D:2 * D].T
    w1b_col = glob_fc1_w[:, 2 * D]
    b1 = glob_fc1_b.reshape(1, D)
    w2xT = glob_fc2_w[:, :D].T
    w2nT = glob_fc2_w[:, D:2 * D].T
    w2aT = glob_fc2_w[:, 2 * D:2 * D + H].T
    w2bT = glob_fc2_w[:, 2 * D + H:].T
    b2 = glob_fc2_b.reshape(1, D)

    grow = group_edge_ids[0].astype(jnp.int32).reshape(EG, 1)
    gcol = group_edge_ids[1].astype(jnp.int32).reshape(EG, 1)

    row = edge_ids[0].astype(jnp.int32)
    col = edge_ids[1].astype(jnp.int32)
    ewi = lax.bitcast_convert_type(edge_weights.reshape(E,).astype(f32),
                                   jnp.int32)
    ice = jnp.stack([
        jnp.stack([row, col, ewi]),
        jnp.stack([row + N, col, ewi]),
    ])
    ice = ice.reshape(NC, 3, NS, NCHUNK, CHUNK, B).transpose(0, 2, 3, 4, 1, 5)
    w1b2 = w1b_col.reshape(NC, H)

    x, gx0 = _call_a(input, embT, embb, tar_group_weights)
    gx = _call_b(gx0, grow, gcol, group_edge_weights, g1xT, g1eT, g1b,
                 g2xT, g2aT, g2b)
    y2, zpart = _call_c(x, tar_group_weights, gx, w1xT, w1nT, b1,
                        w2xT, w2nT, b2)
    ytab = y2.reshape(NC * N, H)
    acc = _edge_call(ytab, ice, w1b2)
    acc, _, _, _ = lax.optimization_barrier((acc, ytab, ice, w1b2))
    out = _call_e(zpart, acc[0, :, :H], acc[1, :, :H],
                  acc[0, :, H:H + 1], w2aT, w2bT)
    return out


# speedup vs baseline: 4.8159x; 2.2880x over previous
import functools

import jax
import jax.numpy as jnp
from jax import lax
from jax.experimental import pallas as pl
from jax.experimental.pallas import tpu as pltpu
from jax.experimental.pallas import tpu_sc as plsc

N = 10000
E = 160000
G = 64
EG = 2048
D = 256
H = 128
ACCW = 144

NC = 2
NS = 16
L = 16

EPT = E // NS
B = 50
NBLK = EPT // B
CHUNK = 20
NCHUNK = NBLK // CHUNK
EU = 5
RPT = N // NS

BN = 1000
NB = N // BN

_F32 = jnp.float32


def _a_body(inp_ref, embT_ref, embb_ref, tgw_ref, x_ref, gacc_ref):
    x = jnp.dot(inp_ref[...], embT_ref[...], preferred_element_type=_F32)
    x = x + embb_ref[...]
    x_ref[...] = x
    part = lax.dot_general(tgw_ref[...], x, (((0,), (0,)), ((), ())),
                           preferred_element_type=_F32)
    @pl.when(pl.program_id(0) == 0)
    def _():
        gacc_ref[...] = part
    @pl.when(pl.program_id(0) != 0)
    def _():
        gacc_ref[...] += part


def _call_a(inp, embT, embb, tgw):
    return pl.pallas_call(
        _a_body,
        grid=(NB,),
        in_specs=[
            pl.BlockSpec((BN, D), lambda i: (i, 0)),
            pl.BlockSpec((D, D), lambda i: (0, 0)),
            pl.BlockSpec((1, D), lambda i: (0, 0)),
            pl.BlockSpec((BN, G), lambda i: (i, 0)),
        ],
        out_specs=[
            pl.BlockSpec((BN, D), lambda i: (i, 0)),
            pl.BlockSpec((G, D), lambda i: (0, 0)),
        ],
        out_shape=[
            jax.ShapeDtypeStruct((N, D), _F32),
            jax.ShapeDtypeStruct((G, D), _F32),
        ],
    )(inp, embT, embb, tgw)


def _b_body(gx0_ref, grow_ref, gcol_ref, gew_ref, g1xT_ref, g1eT_ref,
            g1b_ref, g2xT_ref, g2aT_ref, g2b_ref, gx_ref):
    gx0 = gx0_ref[...]
    ids = lax.broadcasted_iota(jnp.int32, (EG, G), 1)
    oh_r = (grow_ref[...] == ids).astype(_F32)
    oh_c = (gcol_ref[...] == ids).astype(_F32)
    base = jnp.dot(gx0, g1xT_ref[...], preferred_element_type=_F32)
    m = jnp.dot(oh_r, base, preferred_element_type=_F32)
    m = m + jnp.dot(gew_ref[...], g1eT_ref[...], preferred_element_type=_F32)
    m = jnp.maximum(m + g1b_ref[...], 0.0)
    sums = lax.dot_general(oh_c, m, (((0,), (0,)), ((), ())),
                           preferred_element_type=_F32)
    cnt = jnp.sum(oh_c, axis=0, keepdims=True)
    agg = sums / jnp.maximum(cnt, 1.0).reshape(G, 1)
    h = jnp.dot(gx0, g2xT_ref[...], preferred_element_type=_F32)
    h = h + jnp.dot(agg, g2aT_ref[...], preferred_element_type=_F32)
    gx_ref[...] = jnp.maximum(h + g2b_ref[...], 0.0)


def _call_b(gx0, grow, gcol, gew, g1xT, g1eT, g1b, g2xT, g2aT, g2b):
    return pl.pallas_call(
        _b_body,
        out_shape=jax.ShapeDtypeStruct((G, D), _F32),
    )(gx0, grow, gcol, gew, g1xT, g1eT, g1b, g2xT, g2aT, g2b)


def _c_body(x_ref, tgw_ref, gx_ref, w1xT_ref, w1nT_ref, b1_ref, y2_ref):
    x = x_ref[...]
    new_x = jnp.dot(tgw_ref[...], gx_ref[...], preferred_element_type=_F32)
    y = jnp.dot(x, w1xT_ref[...], preferred_element_type=_F32)
    y = y + jnp.dot(new_x, w1nT_ref[...], preferred_element_type=_F32)
    y = y + b1_ref[...]
    y2_ref[0] = y[:, :H]
    y2_ref[1] = y[:, H:]


def _call_c(x, tgw, gx, w1xT, w1nT, b1):
    return pl.pallas_call(
        _c_body,
        grid=(NB,),
        in_specs=[
            pl.BlockSpec((BN, D), lambda i: (i, 0)),
            pl.BlockSpec((BN, G), lambda i: (i, 0)),
            pl.BlockSpec((G, D), lambda i: (0, 0)),
            pl.BlockSpec((D, D), lambda i: (0, 0)),
            pl.BlockSpec((D, D), lambda i: (0, 0)),
            pl.BlockSpec((1, D), lambda i: (0, 0)),
        ],
        out_specs=pl.BlockSpec((2, BN, H), lambda i: (0, i, 0)),
        out_shape=jax.ShapeDtypeStruct((2, N, H), _F32),
    )(x, tgw, gx, w1xT, w1nT, b1)


def _edge_body(ytab, ice, ew3, w1b2, acc_out,
               ibuf, ewbuf, gbuf, obuf, w1b_v, acc_sh, g0, g1, s0m, s1m):
    c = lax.axis_index("c")
    s = lax.axis_index("s")

    pltpu.sync_copy(w1b2.at[c], w1b_v)

    zero = jnp.zeros((L,), _F32)

    def _zrow(e, carry):
        for f in range(ACCW // L):
            obuf[0, e, pl.ds(L * f, L)] = zero
        return carry

    lax.fori_loop(0, B, _zrow, 0)
    nfull = RPT // B
    for k in range(nfull):
        pltpu.sync_copy(obuf.at[0], acc_sh.at[pl.ds(s * RPT + k * B, B)])
    rem = RPT - nfull * B
    if rem:
        pltpu.sync_copy(obuf.at[0, pl.ds(0, rem)],
                        acc_sh.at[pl.ds(s * RPT + nfull * B, rem)])
    plsc.subcore_barrier()

    w1bs = tuple(w1b_v[pl.ds(L * f, L)] for f in range(H // L))
    lane0 = jnp.where(lax.iota(jnp.int32, L) == 0, 1.0, 0.0).astype(_F32)
    gsems = (g0, g1)
    ssems = (s0m, s1m)

    def _issue_g(jrow, slot):
        pltpu.async_copy(ytab.at[ibuf.at[jrow, 0]], gbuf.at[slot], gsems[slot])

    def _wait_g(jrow, slot):
        pltpu.make_async_copy(ytab.at[ibuf.at[jrow, 0]], gbuf.at[slot],
                              gsems[slot]).wait()

    def _issue_s(jrow, slot):
        pltpu.async_copy(obuf.at[slot], acc_sh.at[ibuf.at[jrow, 1]],
                         ssems[slot], add=True)

    def _wait_s(jrow, slot):
        pltpu.make_async_copy(obuf.at[slot], acc_sh.at[ibuf.at[jrow, 1]],
                              ssems[slot]).wait()

    def _compute(jj, slot):
        def _edge(e):
            ew_vec = plsc.load_gather(
                ewbuf, [jnp.full((L,), jj, jnp.int32),
                        jnp.full((L,), e, jnp.int32)])
            for f in range(H // L):
                g = gbuf[slot, e, pl.ds(L * f, L)]
                obuf[slot, e, pl.ds(L * f, L)] = jnp.maximum(
                    g + ew_vec * w1bs[f], 0.0)
            obuf[slot, e, pl.ds(H, L)] = lane0

        plsc.parallel_loop(0, B, 1, unroll=EU)(_edge)

    def _chunk(ch, carry):
        pltpu.sync_copy(ice.at[c, s, ch], ibuf)
        pltpu.sync_copy(ew3.at[s, ch], ewbuf)
        _issue_g(0, 0)

        def _pair(jj2, pcarry):
            j0 = jj2 * 2
            j1 = j0 + 1
            _issue_g(j1, 1)
            _wait_g(j0, 0)

            @pl.when(jj2 > 0)
            def _():
                _wait_s(j0, 0)

            _compute(j0, 0)
            _issue_s(j0, 0)

            @pl.when(j1 + 1 < CHUNK)
            def _():
                _issue_g(j1 + 1, 0)

            _wait_g(j1, 1)

            @pl.when(jj2 > 0)
            def _():
                _wait_s(j1, 1)

            _compute(j1, 1)
            _issue_s(j1, 1)
            return pcarry

        lax.fori_loop(0, CHUNK // 2, _pair, 0)
        _wait_s(CHUNK - 2, 0)
        _wait_s(CHUNK - 1, 1)
        return carry

    lax.fori_loop(0, NCHUNK, _chunk, 0)
    plsc.subcore_barrier()
    pltpu.sync_copy(acc_sh.at[pl.ds(s * RPT, RPT)],
                    acc_out.at[c, pl.ds(s * RPT, RPT)])


def _edge_call(ytab, ice, ew3, w1b2):
    mesh = plsc.VectorSubcoreMesh(core_axis_name="c", subcore_axis_name="s",
                                  num_cores=NC, num_subcores=NS)
    return pl.kernel(
        _edge_body,
        out_type=jax.ShapeDtypeStruct((NC, N, ACCW), _F32),
        mesh=mesh,
        scratch_types=[
            pltpu.VMEM((CHUNK, 2, B), jnp.int32),
            pltpu.VMEM((CHUNK, B), _F32),
            pltpu.VMEM((2, B, H), _F32),
            pltpu.VMEM((2, B, ACCW), _F32),
            pltpu.VMEM((H,), _F32),
            pltpu.VMEM_SHARED((N, ACCW), _F32),
            pltpu.SemaphoreType.DMA,
            pltpu.SemaphoreType.DMA,
            pltpu.SemaphoreType.DMA,
            pltpu.SemaphoreType.DMA,
        ],
        compiler_params=pltpu.CompilerParams(use_tc_tiling_on_sc=False,
                                             needs_layout_passes=False),
    )(ytab, ice, ew3, w1b2)


def _e_body(x_ref, tgw_ref, gx_ref, s0_ref, s1_ref, cnt_ref,
            w2xT_ref, w2nT_ref, b2_ref, w2aT_ref, w2bT_ref, out_ref):
    x = x_ref[...]
    new_x = jnp.dot(tgw_ref[...], gx_ref[...], preferred_element_type=_F32)
    cnt = jnp.maximum(cnt_ref[...], 1.0)
    a0 = s0_ref[...] / cnt
    a1 = s1_ref[...] / cnt
    o = jnp.dot(x, w2xT_ref[...], preferred_element_type=_F32)
    o = o + jnp.dot(new_x, w2nT_ref[...], preferred_element_type=_F32)
    o = o + jnp.dot(a0, w2aT_ref[...], preferred_element_type=_F32)
    o = o + jnp.dot(a1, w2bT_ref[...], preferred_element_type=_F32)
    out_ref[...] = jnp.maximum(o + b2_ref[...], 0.0)


def _call_e(x, tgw, gx, s0, s1, cnt, w2xT, w2nT, b2, w2aT, w2bT):
    return pl.pallas_call(
        _e_body,
        grid=(NB,),
        in_specs=[
            pl.BlockSpec((BN, D), lambda i: (i, 0)),
            pl.BlockSpec((BN, G), lambda i: (i, 0)),
            pl.BlockSpec((G, D), lambda i: (0, 0)),
            pl.BlockSpec((BN, H), lambda i: (i, 0)),
            pl.BlockSpec((BN, H), lambda i: (i, 0)),
            pl.BlockSpec((BN, 1), lambda i: (i, 0)),
            pl.BlockSpec((D, D), lambda i: (0, 0)),
            pl.BlockSpec((D, D), lambda i: (0, 0)),
            pl.BlockSpec((1, D), lambda i: (0, 0)),
            pl.BlockSpec((H, D), lambda i: (0, 0)),
            pl.BlockSpec((H, D), lambda i: (0, 0)),
        ],
        out_specs=pl.BlockSpec((BN, D), lambda i: (i, 0)),
        out_shape=jax.ShapeDtypeStruct((N, D), _F32),
    )(x, tgw, gx, s0, s1, cnt, w2xT, w2nT, b2, w2aT, w2bT)


def kernel(input, tar_group_weights, enc_weights, group_edge_ids,
           group_edge_weights, edge_ids, edge_weights, emb_w, emb_b,
           g_fc1_w, g_fc1_b, g_fc2_w, g_fc2_b, glob_fc1_w, glob_fc1_b,
           glob_fc2_w, glob_fc2_b):
    f32 = _F32
    embT = emb_w.T
    embb = emb_b.reshape(1, D)
    g1xT = g_fc1_w[:, :D].T
    g1eT = g_fc1_w[:, D:].T
    g1b = g_fc1_b.reshape(1, D)
    g2xT = g_fc2_w[:, :D].T
    g2aT = g_fc2_w[:, D:].T
    g2b = g_fc2_b.reshape(1, D)
    w1xT = glob_fc1_w[:, :D].T
    w1nT = glob_fc1_w[:, D:2 * D].T
    w1b_col = glob_fc1_w[:, 2 * D]
    b1 = glob_fc1_b.reshape(1, D)
    w2xT = glob_fc2_w[:, :D].T
    w2nT = glob_fc2_w[:, D:2 * D].T
    w2aT = glob_fc2_w[:, 2 * D:2 * D + H].T
    w2bT = glob_fc2_w[:, 2 * D + H:].T
    b2 = glob_fc2_b.reshape(1, D)

    grow = group_edge_ids[0].astype(jnp.int32).reshape(EG, 1)
    gcol = group_edge_ids[1].astype(jnp.int32).reshape(EG, 1)

    row = edge_ids[0].astype(jnp.int32)
    col = edge_ids[1].astype(jnp.int32)
    ice = jnp.stack([
        jnp.stack([row, col]),
        jnp.stack([row + N, col]),
    ])
    ice = ice.reshape(NC, 2, NS, NCHUNK, CHUNK, B).transpose(0, 2, 3, 4, 1, 5)
    ew3 = edge_weights.reshape(E,).astype(f32).reshape(NS, NCHUNK, CHUNK, B)
    w1b2 = w1b_col.reshape(NC, H)

    x, gx0 = _call_a(input, embT, embb, tar_group_weights)
    gx = _call_b(gx0, grow, gcol, group_edge_weights, g1xT, g1eT, g1b,
                 g2xT, g2aT, g2b)
    y2 = _call_c(x, tar_group_weights, gx, w1xT, w1nT, b1)
    ytab = y2.reshape(NC * N, H)
    acc = _edge_call(ytab, ice, ew3, w1b2)
    acc, _, _, _, _ = lax.optimization_barrier((acc, ytab, ice, ew3, w1b2))
    out = _call_e(x, tar_group_weights, gx, acc[0, :, :H], acc[1, :, :H],
                  acc[0, :, H:H + 1], w2xT, w2nT, b2, w2aT, w2bT)
    return out


# speedup vs baseline: 4.8673x; 1.0107x over previous
import functools

import numpy as np
import jax
import jax.numpy as jnp
from jax import lax
from jax.experimental import pallas as pl
from jax.experimental.pallas import tpu as pltpu
from jax.experimental.pallas import tpu_sc as plsc

N = 10000
E = 160000
G = 64
EG = 2048
EDIM = 16
D = 256
H = 128
ACCW = 144

NC = 2
NS = 16
L = 16

EPT = E // NS
B = 50
NBLK = EPT // B
CHUNK = 20
NCHUNK = NBLK // CHUNK
EU = 5
RPT = N // NS

BN = 1000
NB = N // BN

_PERM = np.concatenate([
    np.concatenate([32 * q + np.arange(0, 32, 2),
                    32 * q + np.arange(1, 32, 2)])
    for q in range(H // 32)
])

_F32 = jnp.float32


def _a_body(inp_ref, embT_ref, embb_ref, tgw_ref, grow_ref, gcol_ref,
            gew_ref, g1xT_ref, g1eT_ref, g1b_ref, g2xT_ref, g2aT_ref,
            g2b_ref, x_ref, gx_ref, gacc_ref):
    x = jnp.dot(inp_ref[...], embT_ref[...], preferred_element_type=_F32)
    x = x + embb_ref[...]
    x_ref[...] = x
    part = lax.dot_general(tgw_ref[...], x, (((0,), (0,)), ((), ())),
                           preferred_element_type=_F32)
    @pl.when(pl.program_id(0) == 0)
    def _():
        gacc_ref[...] = part
    @pl.when(pl.program_id(0) != 0)
    def _():
        gacc_ref[...] += part

    @pl.when(pl.program_id(0) == NB - 1)
    def _():
        gx0 = gacc_ref[...]
        ids = lax.broadcasted_iota(jnp.int32, (EG, G), 1)
        oh_r = (grow_ref[...] == ids).astype(_F32)
        oh_c = (gcol_ref[...] == ids).astype(_F32)
        base = jnp.dot(gx0, g1xT_ref[...], preferred_element_type=_F32)
        m = jnp.dot(oh_r, base, preferred_element_type=_F32)
        m = m + jnp.dot(gew_ref[...], g1eT_ref[...],
                        preferred_element_type=_F32)
        m = jnp.maximum(m + g1b_ref[...], 0.0)
        sums = lax.dot_general(oh_c, m, (((0,), (0,)), ((), ())),
                               preferred_element_type=_F32)
        cnt = jnp.sum(oh_c, axis=0, keepdims=True)
        agg = sums / jnp.maximum(cnt, 1.0).reshape(G, 1)
        h = jnp.dot(gx0, g2xT_ref[...], preferred_element_type=_F32)
        h = h + jnp.dot(agg, g2aT_ref[...], preferred_element_type=_F32)
        gx_ref[...] = jnp.maximum(h + g2b_ref[...], 0.0)


def _call_a(inp, embT, embb, tgw, grow, gcol, gew, g1xT, g1eT, g1b,
            g2xT, g2aT, g2b):
    zero2d = lambda i: (0, 0)
    return pl.pallas_call(
        _a_body,
        grid=(NB,),
        in_specs=[
            pl.BlockSpec((BN, D), lambda i: (i, 0)),
            pl.BlockSpec((D, D), zero2d),
            pl.BlockSpec((1, D), zero2d),
            pl.BlockSpec((BN, G), lambda i: (i, 0)),
            pl.BlockSpec((EG, 1), zero2d),
            pl.BlockSpec((EG, 1), zero2d),
            pl.BlockSpec((EG, EDIM), zero2d),
            pl.BlockSpec((D, D), zero2d),
            pl.BlockSpec((EDIM, D), zero2d),
            pl.BlockSpec((1, D), zero2d),
            pl.BlockSpec((D, D), zero2d),
            pl.BlockSpec((D, D), zero2d),
            pl.BlockSpec((1, D), zero2d),
        ],
        out_specs=[
            pl.BlockSpec((BN, D), lambda i: (i, 0)),
            pl.BlockSpec((G, D), zero2d),
            pl.BlockSpec((G, D), zero2d),
        ],
        out_shape=[
            jax.ShapeDtypeStruct((N, D), _F32),
            jax.ShapeDtypeStruct((G, D), _F32),
            jax.ShapeDtypeStruct((G, D), _F32),
        ],
    )(inp, embT, embb, tgw, grow, gcol, gew, g1xT, g1eT, g1b,
      g2xT, g2aT, g2b)


def _c_body(x_ref, tgw_ref, gx_ref, w1xT_ref, w1nT_ref, b1_ref, y2_ref):
    x = x_ref[...]
    new_x = jnp.dot(tgw_ref[...], gx_ref[...], preferred_element_type=_F32)
    y = jnp.dot(x, w1xT_ref[...], preferred_element_type=_F32)
    y = y + jnp.dot(new_x, w1nT_ref[...], preferred_element_type=_F32)
    y = y + b1_ref[...]
    y16 = y.astype(jnp.bfloat16)
    y2_ref[0] = y16[:, :H]
    y2_ref[1] = y16[:, H:]


def _call_c(x, tgw, gx, w1xT, w1nT, b1):
    return pl.pallas_call(
        _c_body,
        grid=(NB,),
        in_specs=[
            pl.BlockSpec((BN, D), lambda i: (i, 0)),
            pl.BlockSpec((BN, G), lambda i: (i, 0)),
            pl.BlockSpec((G, D), lambda i: (0, 0)),
            pl.BlockSpec((D, D), lambda i: (0, 0)),
            pl.BlockSpec((D, D), lambda i: (0, 0)),
            pl.BlockSpec((1, D), lambda i: (0, 0)),
        ],
        out_specs=pl.BlockSpec((2, BN, H), lambda i: (0, i, 0)),
        out_shape=jax.ShapeDtypeStruct((2, N, H), jnp.bfloat16),
    )(x, tgw, gx, w1xT, w1nT, b1)


def _edge_body(ytab, ice, ew3, w1b2, acc_out,
               ibuf, ewbuf, gbuf, obuf, w1b_v, acc_sh, g0, g1, s0m, s1m):
    c = lax.axis_index("c")
    s = lax.axis_index("s")

    pltpu.sync_copy(w1b2.at[c], w1b_v)

    zero = jnp.zeros((L,), _F32)

    def _zrow(e, carry):
        for f in range(ACCW // L):
            obuf[0, e, pl.ds(L * f, L)] = zero
        return carry

    lax.fori_loop(0, B, _zrow, 0)
    nfull = RPT // B
    for k in range(nfull):
        pltpu.sync_copy(obuf.at[0], acc_sh.at[pl.ds(s * RPT + k * B, B)])
    rem = RPT - nfull * B
    if rem:
        pltpu.sync_copy(obuf.at[0, pl.ds(0, rem)],
                        acc_sh.at[pl.ds(s * RPT + nfull * B, rem)])
    plsc.subcore_barrier()

    w1bs = tuple(w1b_v[pl.ds(L * f, L)] for f in range(H // L))
    lane0 = jnp.where(lax.iota(jnp.int32, L) == 0, 1.0, 0.0).astype(_F32)
    gsems = (g0, g1)
    ssems = (s0m, s1m)

    def _issue_g(jrow, slot):
        pltpu.async_copy(ytab.at[ibuf.at[jrow, 0]], gbuf.at[slot], gsems[slot])

    def _wait_g(jrow, slot):
        pltpu.make_async_copy(ytab.at[ibuf.at[jrow, 0]], gbuf.at[slot],
                              gsems[slot]).wait()

    def _issue_s(jrow, slot):
        pltpu.async_copy(obuf.at[slot], acc_sh.at[ibuf.at[jrow, 1]],
                         ssems[slot], add=True)

    def _wait_s(jrow, slot):
        pltpu.make_async_copy(obuf.at[slot], acc_sh.at[ibuf.at[jrow, 1]],
                              ssems[slot]).wait()

    def _compute(jj, slot):
        def _edge(e):
            ew_vec = plsc.load_gather(
                ewbuf, [jnp.full((L,), jj, jnp.int32),
                        jnp.full((L,), e, jnp.int32)])
            for q in range(H // (2 * L)):
                ab = gbuf[slot, e, pl.ds(2 * L * q, 2 * L)]
                a, b = plsc.unpack(ab, format=plsc.PackFormat.INTERLEAVED)
                obuf[slot, e, pl.ds(2 * L * q, L)] = jnp.maximum(
                    a + ew_vec * w1bs[2 * q], 0.0)
                obuf[slot, e, pl.ds(2 * L * q + L, L)] = jnp.maximum(
                    b + ew_vec * w1bs[2 * q + 1], 0.0)
            obuf[slot, e, pl.ds(H, L)] = lane0

        plsc.parallel_loop(0, B, 1, unroll=EU)(_edge)

    def _chunk(ch, carry):
        pltpu.sync_copy(ice.at[c, s, ch], ibuf)
        pltpu.sync_copy(ew3.at[s, ch], ewbuf)
        _issue_g(0, 0)

        def _pair(jj2, pcarry):
            j0 = jj2 * 2
            j1 = j0 + 1
            _issue_g(j1, 1)
            _wait_g(j0, 0)

            @pl.when(jj2 > 0)
            def _():
                _wait_s(j0, 0)

            _compute(j0, 0)
            _issue_s(j0, 0)

            @pl.when(j1 + 1 < CHUNK)
            def _():
                _issue_g(j1 + 1, 0)

            _wait_g(j1, 1)

            @pl.when(jj2 > 0)
            def _():
                _wait_s(j1, 1)

            _compute(j1, 1)
            _issue_s(j1, 1)
            return pcarry

        lax.fori_loop(0, CHUNK // 2, _pair, 0)
        _wait_s(CHUNK - 2, 0)
        _wait_s(CHUNK - 1, 1)
        return carry

    lax.fori_loop(0, NCHUNK, _chunk, 0)
    plsc.subcore_barrier()
    pltpu.sync_copy(acc_sh.at[pl.ds(s * RPT, RPT)],
                    acc_out.at[c, pl.ds(s * RPT, RPT)])


def _edge_call(ytab, ice, ew3, w1b2):
    mesh = plsc.VectorSubcoreMesh(core_axis_name="c", subcore_axis_name="s",
                                  num_cores=NC, num_subcores=NS)
    return pl.kernel(
        _edge_body,
        out_type=jax.ShapeDtypeStruct((NC, N, ACCW), _F32),
        mesh=mesh,
        scratch_types=[
            pltpu.VMEM((CHUNK, 2, B), jnp.int32),
            pltpu.VMEM((CHUNK, B), _F32),
            pltpu.VMEM((2, B, H), jnp.bfloat16),
            pltpu.VMEM((2, B, ACCW), _F32),
            pltpu.VMEM((H,), _F32),
            pltpu.VMEM_SHARED((N, ACCW), _F32),
            pltpu.SemaphoreType.DMA,
            pltpu.SemaphoreType.DMA,
            pltpu.SemaphoreType.DMA,
            pltpu.SemaphoreType.DMA,
        ],
        compiler_params=pltpu.CompilerParams(use_tc_tiling_on_sc=False,
                                             needs_layout_passes=False),
    )(ytab, ice, ew3, w1b2)


def _e_body(x_ref, tgw_ref, gx_ref, s0_ref, s1_ref, cnt_ref,
            w2xT_ref, w2nT_ref, b2_ref, w2aT_ref, w2bT_ref, out_ref):
    x = x_ref[...]
    new_x = jnp.dot(tgw_ref[...], gx_ref[...], preferred_element_type=_F32)
    cnt = jnp.maximum(cnt_ref[...], 1.0)
    a0 = s0_ref[0] / cnt
    a1 = s1_ref[0] / cnt
    o = jnp.dot(x, w2xT_ref[...], preferred_element_type=_F32)
    o = o + jnp.dot(new_x, w2nT_ref[...], preferred_element_type=_F32)
    o = o + jnp.dot(a0, w2aT_ref[...], preferred_element_type=_F32)
    o = o + jnp.dot(a1, w2bT_ref[...], preferred_element_type=_F32)
    out_ref[...] = jnp.maximum(o + b2_ref[...], 0.0)


def _call_e(x, tgw, gx, acc, cnt, w2xT, w2nT, b2, w2aT, w2bT):
    return pl.pallas_call(
        _e_body,
        grid=(NB,),
        in_specs=[
            pl.BlockSpec((BN, D), lambda i: (i, 0)),
            pl.BlockSpec((BN, G), lambda i: (i, 0)),
            pl.BlockSpec((G, D), lambda i: (0, 0)),
            pl.BlockSpec((1, BN, H), lambda i: (0, i, 0)),
            pl.BlockSpec((1, BN, H), lambda i: (1, i, 0)),
            pl.BlockSpec((BN, 1), lambda i: (i, 0)),
            pl.BlockSpec((D, D), lambda i: (0, 0)),
            pl.BlockSpec((D, D), lambda i: (0, 0)),
            pl.BlockSpec((1, D), lambda i: (0, 0)),
            pl.BlockSpec((H, D), lambda i: (0, 0)),
            pl.BlockSpec((H, D), lambda i: (0, 0)),
        ],
        out_specs=pl.BlockSpec((BN, D), lambda i: (i, 0)),
        out_shape=jax.ShapeDtypeStruct((N, D), _F32),
    )(x, tgw, gx, acc, acc, cnt, w2xT, w2nT, b2, w2aT, w2bT)


def kernel(input, tar_group_weights, enc_weights, group_edge_ids,
           group_edge_weights, edge_ids, edge_weights, emb_w, emb_b,
           g_fc1_w, g_fc1_b, g_fc2_w, g_fc2_b, glob_fc1_w, glob_fc1_b,
           glob_fc2_w, glob_fc2_b):
    f32 = _F32
    embT = emb_w.T
    embb = emb_b.reshape(1, D)
    g1xT = g_fc1_w[:, :D].T
    g1eT = g_fc1_w[:, D:].T
    g1b = g_fc1_b.reshape(1, D)
    g2xT = g_fc2_w[:, :D].T
    g2aT = g_fc2_w[:, D:].T
    g2b = g_fc2_b.reshape(1, D)
    w1xT = glob_fc1_w[:, :D].T
    w1nT = glob_fc1_w[:, D:2 * D].T
    w1b_col = glob_fc1_w[:, 2 * D]
    b1 = glob_fc1_b.reshape(1, D)
    w2xT = glob_fc2_w[:, :D].T
    w2nT = glob_fc2_w[:, D:2 * D].T
    w2aT = glob_fc2_w[:, 2 * D:2 * D + H].T[_PERM, :]
    w2bT = glob_fc2_w[:, 2 * D + H:].T[_PERM, :]
    b2 = glob_fc2_b.reshape(1, D)

    grow = group_edge_ids[0].astype(jnp.int32).reshape(EG, 1)
    gcol = group_edge_ids[1].astype(jnp.int32).reshape(EG, 1)

    row = edge_ids[0].astype(jnp.int32)
    col = edge_ids[1].astype(jnp.int32)
    ice = jnp.stack([
        jnp.stack([row, col]),
        jnp.stack([row + N, col]),
    ])
    ice = ice.reshape(NC, 2, NS, NCHUNK, CHUNK, B).transpose(0, 2, 3, 4, 1, 5)
    ew3 = edge_weights.reshape(E,).astype(f32).reshape(NS, NCHUNK, CHUNK, B)
    w1b2 = w1b_col.reshape(NC, H)[:, _PERM]

    x, gx, _ = _call_a(input, embT, embb, tar_group_weights, grow, gcol,
                       group_edge_weights, g1xT, g1eT, g1b, g2xT, g2aT, g2b)
    y2 = _call_c(x, tar_group_weights, gx, w1xT, w1nT, b1)
    ytab = y2.reshape(NC * N, H)
    acc = _edge_call(ytab, ice, ew3, w1b2)
    acc, _, _, _, _ = lax.optimization_barrier((acc, ytab, ice, ew3, w1b2))
    out = _call_e(x, tar_group_weights, gx, acc, acc[0, :, H:H + 1],
                  w2xT, w2nT, b2, w2aT, w2bT)
    return out


# speedup vs baseline: 6.1506x; 1.2637x over previous
import functools

import numpy as np
import jax
import jax.numpy as jnp
from jax import lax
from jax.experimental import pallas as pl
from jax.experimental.pallas import tpu as pltpu
from jax.experimental.pallas import tpu_sc as plsc

N = 10000
E = 160000
G = 64
EG = 2048
EDIM = 16
D = 256
H = 128
CW = 16

NC = 2
NS = 16
L = 16

EPT = E // NS
B = 50
NBLK = EPT // B
CHUNK = 20
NCHUNK = NBLK // CHUNK
EU = 5
RPT = N // NS

BN = 1000
NB = N // BN

_PERM = np.concatenate([
    np.concatenate([32 * q + np.arange(0, 32, 2),
                    32 * q + np.arange(1, 32, 2)])
    for q in range(H // 32)
])

_F32 = jnp.float32


_CT11 = (((1,), (1,)), ((), ()))


def _a_body(inp_ref, emb_ref, embb_ref, tgw_ref, grow_ref, gcol_ref,
            gew_ref, g1x_ref, g1e_ref, g1b_ref, g2x_ref, g2a_ref,
            g2b_ref, x_ref, gx_ref, gacc_ref):
    x = lax.dot_general(inp_ref[...], emb_ref[...], _CT11,
                        preferred_element_type=_F32)
    x = x + embb_ref[...]
    x_ref[...] = x
    part = lax.dot_general(tgw_ref[...], x, (((0,), (0,)), ((), ())),
                           preferred_element_type=_F32)
    @pl.when(pl.program_id(0) == 0)
    def _():
        gacc_ref[...] = part
    @pl.when(pl.program_id(0) != 0)
    def _():
        gacc_ref[...] += part

    @pl.when(pl.program_id(0) == NB - 1)
    def _():
        gx0 = gacc_ref[...]
        ids = lax.broadcasted_iota(jnp.int32, (EG, G), 1)
        oh_r = (grow_ref[...] == ids).astype(_F32)
        oh_c = (gcol_ref[...] == ids).astype(_F32)
        base = lax.dot_general(gx0, g1x_ref[...], _CT11,
                               preferred_element_type=_F32)
        m = jnp.dot(oh_r, base, preferred_element_type=_F32)
        m = m + lax.dot_general(gew_ref[...], g1e_ref[...], _CT11,
                                preferred_element_type=_F32)
        m = jnp.maximum(m + g1b_ref[...], 0.0)
        sums = lax.dot_general(oh_c, m, (((0,), (0,)), ((), ())),
                               preferred_element_type=_F32)
        cnt = jnp.sum(oh_c, axis=0, keepdims=True)
        agg = sums / jnp.maximum(cnt, 1.0).reshape(G, 1)
        h = lax.dot_general(gx0, g2x_ref[...], _CT11,
                            preferred_element_type=_F32)
        h = h + lax.dot_general(agg, g2a_ref[...], _CT11,
                                preferred_element_type=_F32)
        gx_ref[...] = jnp.maximum(h + g2b_ref[...], 0.0)


def _call_a(inp, emb_w, embb, tgw, grow, gcol, gew, g1w, g1e, g1b,
            g2w, g2b):
    zero2d = lambda i: (0, 0)
    return pl.pallas_call(
        _a_body,
        grid=(NB,),
        in_specs=[
            pl.BlockSpec((BN, D), lambda i: (i, 0)),
            pl.BlockSpec((D, D), zero2d),
            pl.BlockSpec((1, D), zero2d),
            pl.BlockSpec((BN, G), lambda i: (i, 0)),
            pl.BlockSpec((EG, 1), zero2d),
            pl.BlockSpec((EG, 1), zero2d),
            pl.BlockSpec((EG, EDIM), zero2d),
            pl.BlockSpec((D, D), zero2d),
            pl.BlockSpec((D, EDIM), zero2d),
            pl.BlockSpec((1, D), zero2d),
            pl.BlockSpec((D, D), zero2d),
            pl.BlockSpec((D, D), lambda i: (0, 1)),
            pl.BlockSpec((1, D), zero2d),
        ],
        out_specs=[
            pl.BlockSpec((BN, D), lambda i: (i, 0)),
            pl.BlockSpec((G, D), zero2d),
            pl.BlockSpec((G, D), zero2d),
        ],
        out_shape=[
            jax.ShapeDtypeStruct((N, D), _F32),
            jax.ShapeDtypeStruct((G, D), _F32),
            jax.ShapeDtypeStruct((G, D), _F32),
        ],
    )(inp, emb_w, embb, tgw, grow, gcol, gew, g1w, g1e, g1b,
      g2w, g2w, g2b)


def _c_body(x_ref, tgw_ref, gx_ref, wA_ref, wB_ref, b1_ref,
            yt0_ref, yt1_ref):
    x = x_ref[...]
    new_x = jnp.dot(tgw_ref[...], gx_ref[...], preferred_element_type=_F32)
    y = lax.dot_general(x, wA_ref[...], _CT11, preferred_element_type=_F32)
    y = y + lax.dot_general(new_x, wB_ref[...], _CT11,
                            preferred_element_type=_F32)
    y = y + b1_ref[...]
    y16 = y.astype(jnp.bfloat16)
    yt0_ref[...] = y16[:, :H]
    yt1_ref[...] = y16[:, H:]


def _call_c(x, tgw, gx, w1, b1):
    return pl.pallas_call(
        _c_body,
        grid=(NB,),
        in_specs=[
            pl.BlockSpec((BN, D), lambda i: (i, 0)),
            pl.BlockSpec((BN, G), lambda i: (i, 0)),
            pl.BlockSpec((G, D), lambda i: (0, 0)),
            pl.BlockSpec((D, D), lambda i: (0, 0)),
            pl.BlockSpec((D, D), lambda i: (0, 1)),
            pl.BlockSpec((1, D), lambda i: (0, 0)),
        ],
        out_specs=[
            pl.BlockSpec((BN, H), lambda i: (i, 0)),
            pl.BlockSpec((BN, H), lambda i: (i, 0)),
        ],
        out_shape=[
            jax.ShapeDtypeStruct((N, H), jnp.bfloat16),
            jax.ShapeDtypeStruct((N, H), jnp.bfloat16),
        ],
    )(x, tgw, gx, w1, w1, b1)


def _edge_body(ytab0, ytab1, row4, col4, ew4, w1b2, acc_out, cnt_out,
               rbuf, cbuf, ewbuf, gbuf, obuf, cones, w1b_v, acc_sh, cnt_sh,
               g0, g1, s0m, s1m, csem):
    c = lax.axis_index("c")
    s = lax.axis_index("s")

    pltpu.sync_copy(w1b2.at[c], w1b_v)

    zero = jnp.zeros((L,), _F32)
    lane0 = jnp.where(lax.iota(jnp.int32, L) == 0, 1.0, 0.0).astype(_F32)

    def _zrow(e, carry):
        for f in range(H // L):
            obuf[0, e, pl.ds(L * f, L)] = zero
        cones[e, pl.ds(0, L)] = lane0
        return carry

    lax.fori_loop(0, B, _zrow, 0)
    nfull = RPT // B
    for k in range(nfull):
        pltpu.sync_copy(obuf.at[0], acc_sh.at[pl.ds(s * RPT + k * B, B)])
    rem = RPT - nfull * B
    if rem:
        pltpu.sync_copy(obuf.at[0, pl.ds(0, rem)],
                        acc_sh.at[pl.ds(s * RPT + nfull * B, rem)])

    @pl.when(c == 0)
    def _():
        def _zc(k2, carry):
            pltpu.sync_copy(obuf.at[0, pl.ds(0, B), pl.ds(0, CW)],
                            cnt_sh.at[pl.ds(s * RPT + k2 * B, B)])
            return carry
        lax.fori_loop(0, nfull, _zc, 0)
        if rem:
            pltpu.sync_copy(obuf.at[0, pl.ds(0, rem), pl.ds(0, CW)],
                            cnt_sh.at[pl.ds(s * RPT + nfull * B, rem)])

    plsc.subcore_barrier()

    w1bs = tuple(w1b_v[pl.ds(L * f, L)] for f in range(H // L))
    gsems = (g0, g1)
    ssems = (s0m, s1m)

    def _issue_g(jrow, slot):
        @pl.when(c == 0)
        def _():
            pltpu.async_copy(ytab0.at[rbuf.at[jrow]], gbuf.at[slot],
                             gsems[slot])

        @pl.when(c == 1)
        def _():
            pltpu.async_copy(ytab1.at[rbuf.at[jrow]], gbuf.at[slot],
                             gsems[slot])

    def _wait_g(jrow, slot):
        pltpu.make_async_copy(ytab0.at[rbuf.at[jrow]], gbuf.at[slot],
                              gsems[slot]).wait()

    def _issue_s(jrow, slot):
        pltpu.async_copy(obuf.at[slot], acc_sh.at[cbuf.at[jrow]],
                         ssems[slot], add=True)

    def _wait_s(jrow, slot):
        pltpu.make_async_copy(obuf.at[slot], acc_sh.at[cbuf.at[jrow]],
                              ssems[slot]).wait()

    def _issue_c(jrow):
        pltpu.async_copy(cones, cnt_sh.at[cbuf.at[jrow]], csem, add=True)

    def _wait_c(jrow):
        pltpu.make_async_copy(cones, cnt_sh.at[cbuf.at[jrow]], csem).wait()

    def _compute(jj, slot):
        def _edge(e):
            ew_vec = plsc.load_gather(
                ewbuf, [jnp.full((L,), jj, jnp.int32),
                        jnp.full((L,), e, jnp.int32)])
            for q in range(H // (2 * L)):
                ab = gbuf[slot, e, pl.ds(2 * L * q, 2 * L)]
                a, b = plsc.unpack(ab, format=plsc.PackFormat.INTERLEAVED)
                obuf[slot, e, pl.ds(2 * L * q, L)] = jnp.maximum(
                    a + ew_vec * w1bs[2 * q], 0.0)
                obuf[slot, e, pl.ds(2 * L * q + L, L)] = jnp.maximum(
                    b + ew_vec * w1bs[2 * q + 1], 0.0)

        plsc.parallel_loop(0, B, 1, unroll=EU)(_edge)

    def _chunk(ch, carry):
        pltpu.sync_copy(row4.at[s, ch], rbuf)
        pltpu.sync_copy(col4.at[s, ch], cbuf)
        pltpu.sync_copy(ew4.at[s, ch], ewbuf)
        _issue_g(0, 0)

        def _pair(jj2, pcarry):
            j0 = jj2 * 2
            j1 = j0 + 1
            _issue_g(j1, 1)

            @pl.when(c == 0)
            def _():
                _issue_c(j0)
                _issue_c(j1)

            _wait_g(j0, 0)

            @pl.when(jj2 > 0)
            def _():
                _wait_s(j0, 0)

            _compute(j0, 0)
            _issue_s(j0, 0)

            @pl.when(j1 + 1 < CHUNK)
            def _():
                _issue_g(j1 + 1, 0)

            _wait_g(j1, 1)

            @pl.when(jj2 > 0)
            def _():
                _wait_s(j1, 1)

            _compute(j1, 1)
            _issue_s(j1, 1)
            return pcarry

        lax.fori_loop(0, CHUNK // 2, _pair, 0)
        _wait_s(CHUNK - 2, 0)
        _wait_s(CHUNK - 1, 1)

        @pl.when(c == 0)
        def _():
            def _drain(jd, dcarry):
                _wait_c(jd)
                return dcarry
            lax.fori_loop(0, CHUNK, _drain, 0)

        return carry

    lax.fori_loop(0, NCHUNK, _chunk, 0)
    plsc.subcore_barrier()
    pltpu.sync_copy(acc_sh.at[pl.ds(s * RPT, RPT)],
                    acc_out.at[c, pl.ds(s * RPT, RPT)])

    @pl.when(c == 0)
    def _():
        pltpu.sync_copy(cnt_sh.at[pl.ds(s * RPT, RPT)],
                        cnt_out.at[pl.ds(s * RPT, RPT)])


def _edge_call(ytab0, ytab1, row4, col4, ew4, w1b2):
    mesh = plsc.VectorSubcoreMesh(core_axis_name="c", subcore_axis_name="s",
                                  num_cores=NC, num_subcores=NS)
    return pl.kernel(
        _edge_body,
        out_type=[
            jax.ShapeDtypeStruct((NC, N, H), _F32),
            jax.ShapeDtypeStruct((N, CW), _F32),
        ],
        mesh=mesh,
        scratch_types=[
            pltpu.VMEM((CHUNK, B), jnp.int32),
            pltpu.VMEM((CHUNK, B), jnp.int32),
            pltpu.VMEM((CHUNK, B), _F32),
            pltpu.VMEM((2, B, H), jnp.bfloat16),
            pltpu.VMEM((2, B, H), _F32),
            pltpu.VMEM((B, CW), _F32),
            pltpu.VMEM((H,), _F32),
            pltpu.VMEM_SHARED((N, H), _F32),
            pltpu.VMEM_SHARED((N, CW), _F32),
            pltpu.SemaphoreType.DMA,
            pltpu.SemaphoreType.DMA,
            pltpu.SemaphoreType.DMA,
            pltpu.SemaphoreType.DMA,
            pltpu.SemaphoreType.DMA,
        ],
        compiler_params=pltpu.CompilerParams(use_tc_tiling_on_sc=False,
                                             needs_layout_passes=False),
    )(ytab0, ytab1, row4, col4, ew4, w1b2)


def _e_body(x_ref, tgw_ref, gx_ref, a0_ref, a1_ref, cnt_ref,
            w2x_ref, w2n_ref, b2_ref, w2aT_ref, w2bT_ref, out_ref):
    x = x_ref[...]
    new_x = jnp.dot(tgw_ref[...], gx_ref[...], preferred_element_type=_F32)
    cnt = jnp.maximum(cnt_ref[:, :1], 1.0)
    a0 = a0_ref[0] / cnt
    a1 = a1_ref[0] / cnt
    o = lax.dot_general(x, w2x_ref[...], _CT11, preferred_element_type=_F32)
    o = o + lax.dot_general(new_x, w2n_ref[...], _CT11,
                            preferred_element_type=_F32)
    o = o + jnp.dot(a0, w2aT_ref[...], preferred_element_type=_F32)
    o = o + jnp.dot(a1, w2bT_ref[...], preferred_element_type=_F32)
    out_ref[...] = jnp.maximum(o + b2_ref[...], 0.0)


def _call_e(x, tgw, gx, acc, cnt, w2, b2, w2aT, w2bT):
    return pl.pallas_call(
        _e_body,
        grid=(NB,),
        in_specs=[
            pl.BlockSpec((BN, D), lambda i: (i, 0)),
            pl.BlockSpec((BN, G), lambda i: (i, 0)),
            pl.BlockSpec((G, D), lambda i: (0, 0)),
            pl.BlockSpec((1, BN, H), lambda i: (0, i, 0)),
            pl.BlockSpec((1, BN, H), lambda i: (1, i, 0)),
            pl.BlockSpec((BN, CW), lambda i: (i, 0)),
            pl.BlockSpec((D, D), lambda i: (0, 0)),
            pl.BlockSpec((D, D), lambda i: (0, 1)),
            pl.BlockSpec((1, D), lambda i: (0, 0)),
            pl.BlockSpec((H, D), lambda i: (0, 0)),
            pl.BlockSpec((H, D), lambda i: (0, 0)),
        ],
        out_specs=pl.BlockSpec((BN, D), lambda i: (i, 0)),
        out_shape=jax.ShapeDtypeStruct((N, D), _F32),
    )(x, tgw, gx, acc, acc, cnt, w2, w2, b2, w2aT, w2bT)


def kernel(input, tar_group_weights, enc_weights, group_edge_ids,
           group_edge_weights, edge_ids, edge_weights, emb_w, emb_b,
           g_fc1_w, g_fc1_b, g_fc2_w, g_fc2_b, glob_fc1_w, glob_fc1_b,
           glob_fc2_w, glob_fc2_b):
    f32 = _F32
    embb = emb_b.reshape(1, D)
    g1e = g_fc1_w[:, D:]
    g1b = g_fc1_b.reshape(1, D)
    g2b = g_fc2_b.reshape(1, D)
    w1b_col = glob_fc1_w[:, 2 * D]
    b1 = glob_fc1_b.reshape(1, D)
    w2aT = glob_fc2_w[:, 2 * D:2 * D + H].T[_PERM, :]
    w2bT = glob_fc2_w[:, 2 * D + H:].T[_PERM, :]
    b2 = glob_fc2_b.reshape(1, D)

    grow = group_edge_ids[0].astype(jnp.int32).reshape(EG, 1)
    gcol = group_edge_ids[1].astype(jnp.int32).reshape(EG, 1)

    row = edge_ids[0].astype(jnp.int32)
    col = edge_ids[1].astype(jnp.int32)
    row4 = row.reshape(NS, NCHUNK, CHUNK, B)
    col4 = col.reshape(NS, NCHUNK, CHUNK, B)
    ew4 = edge_weights.reshape(E,).astype(f32).reshape(NS, NCHUNK, CHUNK, B)
    w1b2 = w1b_col.reshape(NC, H)[:, _PERM]

    x, gx, _ = _call_a(input, emb_w, embb, tar_group_weights, grow, gcol,
                       group_edge_weights, g_fc1_w, g1e, g1b, g_fc2_w, g2b)
    ytab0, ytab1 = _call_c(x, tar_group_weights, gx, glob_fc1_w, b1)
    acc, cnt = _edge_call(ytab0, ytab1, row4, col4, ew4, w1b2)
    acc, cnt, _, _, _, _, _, _ = lax.optimization_barrier(
        (acc, cnt, ytab0, ytab1, row4, col4, ew4, w1b2))
    out = _call_e(x, tar_group_weights, gx, acc, cnt, glob_fc2_w, b2,
                  w2aT, w2bT)
    return out


# speedup vs baseline: 6.1518x; 1.0002x over previous
import functools

import numpy as np
import jax
import jax.numpy as jnp
from jax import lax
from jax.experimental import pallas as pl
from jax.experimental.pallas import tpu as pltpu
from jax.experimental.pallas import tpu_sc as plsc

N = 10000
E = 160000
G = 64
EG = 2048
EDIM = 16
D = 256
H = 128
CW = 16

NC = 2
NS = 16
L = 16

EPT = E // NS
B = 50
NBLK = EPT // B
CHUNK = 20
NCHUNK = NBLK // CHUNK
EU = 5
RPT = N // NS

BN = 1000
NB = N // BN

_PERM = np.concatenate([
    np.concatenate([32 * q + np.arange(0, 32, 2),
                    32 * q + np.arange(1, 32, 2)])
    for q in range(H // 32)
])

_F32 = jnp.float32


_CT11 = (((1,), (1,)), ((), ()))


def _a_body(inp_ref, emb_ref, embb_ref, tgw_ref, grow_ref, gcol_ref,
            gew_ref, g1x_ref, g1e_ref, g1b_ref, g2x_ref, g2a_ref,
            g2b_ref, x_ref, gx_ref, gacc_ref):
    x = lax.dot_general(inp_ref[...], emb_ref[...], _CT11,
                        preferred_element_type=_F32)
    x = x + embb_ref[...]
    x_ref[...] = x
    part = lax.dot_general(tgw_ref[...], x, (((0,), (0,)), ((), ())),
                           preferred_element_type=_F32)
    @pl.when(pl.program_id(0) == 0)
    def _():
        gacc_ref[...] = part
    @pl.when(pl.program_id(0) != 0)
    def _():
        gacc_ref[...] += part

    @pl.when(pl.program_id(0) == NB - 1)
    def _():
        gx0 = gacc_ref[...]
        ids = lax.broadcasted_iota(jnp.int32, (EG, G), 1)
        oh_r = (grow_ref[...] == ids).astype(_F32)
        oh_c = (gcol_ref[...] == ids).astype(_F32)
        base = lax.dot_general(gx0, g1x_ref[...], _CT11,
                               preferred_element_type=_F32)
        m = jnp.dot(oh_r, base, preferred_element_type=_F32)
        m = m + lax.dot_general(gew_ref[...], g1e_ref[...], _CT11,
                                preferred_element_type=_F32)
        m = jnp.maximum(m + g1b_ref[...], 0.0)
        sums = lax.dot_general(oh_c, m, (((0,), (0,)), ((), ())),
                               preferred_element_type=_F32)
        cnt = jnp.sum(oh_c, axis=0, keepdims=True)
        agg = sums / jnp.maximum(cnt, 1.0).reshape(G, 1)
        h = lax.dot_general(gx0, g2x_ref[...], _CT11,
                            preferred_element_type=_F32)
        h = h + lax.dot_general(agg, g2a_ref[...], _CT11,
                                preferred_element_type=_F32)
        gx_ref[...] = jnp.maximum(h + g2b_ref[...], 0.0)


def _call_a(inp, emb_w, embb, tgw, grow, gcol, gew, g1w, g1e, g1b,
            g2w, g2b):
    zero2d = lambda i: (0, 0)
    return pl.pallas_call(
        _a_body,
        grid=(NB,),
        in_specs=[
            pl.BlockSpec((BN, D), lambda i: (i, 0)),
            pl.BlockSpec((D, D), zero2d),
            pl.BlockSpec((1, D), zero2d),
            pl.BlockSpec((BN, G), lambda i: (i, 0)),
            pl.BlockSpec((EG, 1), zero2d),
            pl.BlockSpec((EG, 1), zero2d),
            pl.BlockSpec((EG, EDIM), zero2d),
            pl.BlockSpec((D, D), zero2d),
            pl.BlockSpec((D, EDIM), zero2d),
            pl.BlockSpec((1, D), zero2d),
            pl.BlockSpec((D, D), zero2d),
            pl.BlockSpec((D, D), lambda i: (0, 1)),
            pl.BlockSpec((1, D), zero2d),
        ],
        out_specs=[
            pl.BlockSpec((BN, D), lambda i: (i, 0)),
            pl.BlockSpec((G, D), zero2d),
            pl.BlockSpec((G, D), zero2d),
        ],
        out_shape=[
            jax.ShapeDtypeStruct((N, D), _F32),
            jax.ShapeDtypeStruct((G, D), _F32),
            jax.ShapeDtypeStruct((G, D), _F32),
        ],
    )(inp, emb_w, embb, tgw, grow, gcol, gew, g1w, g1e, g1b,
      g2w, g2w, g2b)


def _c_body(x_ref, tgw_ref, gx_ref, wA_ref, wB_ref, b1_ref,
            yt0_ref, yt1_ref):
    x = x_ref[...]
    new_x = jnp.dot(tgw_ref[...], gx_ref[...], preferred_element_type=_F32)
    y = lax.dot_general(x, wA_ref[...], _CT11, preferred_element_type=_F32)
    y = y + lax.dot_general(new_x, wB_ref[...], _CT11,
                            preferred_element_type=_F32)
    y = y + b1_ref[...]
    y16 = y.astype(jnp.bfloat16)
    yt0_ref[...] = y16[:, :H]
    yt1_ref[...] = y16[:, H:]


def _call_c(x, tgw, gx, w1, b1):
    return pl.pallas_call(
        _c_body,
        grid=(NB,),
        in_specs=[
            pl.BlockSpec((BN, D), lambda i: (i, 0)),
            pl.BlockSpec((BN, G), lambda i: (i, 0)),
            pl.BlockSpec((G, D), lambda i: (0, 0)),
            pl.BlockSpec((D, D), lambda i: (0, 0)),
            pl.BlockSpec((D, D), lambda i: (0, 1)),
            pl.BlockSpec((1, D), lambda i: (0, 0)),
        ],
        out_specs=[
            pl.BlockSpec((BN, H), lambda i: (i, 0)),
            pl.BlockSpec((BN, H), lambda i: (i, 0)),
        ],
        out_shape=[
            jax.ShapeDtypeStruct((N, H), jnp.bfloat16),
            jax.ShapeDtypeStruct((N, H), jnp.bfloat16),
        ],
    )(x, tgw, gx, w1, w1, b1)


def _edge_body(ytab0, ytab1, row4, col4, ew4, w1b2, acc_out, cnt_out,
               rbuf, cbuf, ewbuf, gbuf, obuf, cones, w1b_v, acc_sh, cnt_sh,
               g0, g1, s0m, s1m, csem):
    c = lax.axis_index("c")
    s = lax.axis_index("s")

    pltpu.sync_copy(w1b2.at[c], w1b_v)

    zero = jnp.zeros((L,), _F32)
    lane0 = jnp.where(lax.iota(jnp.int32, L) == 0, 1.0, 0.0).astype(_F32)

    def _zrow(e, carry):
        for f in range(H // L):
            obuf[0, e, pl.ds(L * f, L)] = zero
        cones[e, pl.ds(0, L)] = lane0
        return carry

    lax.fori_loop(0, B, _zrow, 0)
    nfull = RPT // B
    for k in range(nfull):
        pltpu.sync_copy(obuf.at[0], acc_sh.at[pl.ds(s * RPT + k * B, B)])
    rem = RPT - nfull * B
    if rem:
        pltpu.sync_copy(obuf.at[0, pl.ds(0, rem)],
                        acc_sh.at[pl.ds(s * RPT + nfull * B, rem)])

    @pl.when(c == 0)
    def _():
        def _zc(k2, carry):
            pltpu.sync_copy(obuf.at[0, pl.ds(0, B), pl.ds(0, CW)],
                            cnt_sh.at[pl.ds(s * RPT + k2 * B, B)])
            return carry
        lax.fori_loop(0, nfull, _zc, 0)
        if rem:
            pltpu.sync_copy(obuf.at[0, pl.ds(0, rem), pl.ds(0, CW)],
                            cnt_sh.at[pl.ds(s * RPT + nfull * B, rem)])

    plsc.subcore_barrier()

    w1bs = tuple(w1b_v[pl.ds(L * f, L)] for f in range(H // L))
    gsems = (g0, g1)
    ssems = (s0m, s1m)

    def _issue_g(jrow, slot):
        @pl.when(c == 0)
        def _():
            pltpu.async_copy(ytab0.at[rbuf.at[jrow]], gbuf.at[slot],
                             gsems[slot])

        @pl.when(c == 1)
        def _():
            pltpu.async_copy(ytab1.at[rbuf.at[jrow]], gbuf.at[slot],
                             gsems[slot])

    def _wait_g(jrow, slot):
        @pl.when(c == 0)
        def _():
            pltpu.make_async_copy(ytab0.at[rbuf.at[jrow]], gbuf.at[slot],
                                  gsems[slot]).wait()

        @pl.when(c == 1)
        def _():
            pltpu.make_async_copy(ytab1.at[rbuf.at[jrow]], gbuf.at[slot],
                                  gsems[slot]).wait()

    def _issue_s(jrow, slot):
        pltpu.async_copy(obuf.at[slot], acc_sh.at[cbuf.at[jrow]],
                         ssems[slot], add=True)

    def _wait_s(jrow, slot):
        pltpu.make_async_copy(obuf.at[slot], acc_sh.at[cbuf.at[jrow]],
                              ssems[slot]).wait()

    def _issue_c(jrow):
        pltpu.async_copy(cones, cnt_sh.at[cbuf.at[jrow]], csem, add=True)

    def _wait_c(jrow):
        pltpu.make_async_copy(cones, cnt_sh.at[cbuf.at[jrow]], csem).wait()

    def _compute(jj, slot):
        def _edge(e):
            ew_vec = plsc.load_gather(
                ewbuf, [jnp.full((L,), jj, jnp.int32),
                        jnp.full((L,), e, jnp.int32)])
            for q in range(H // (2 * L)):
                ab = gbuf[slot, e, pl.ds(2 * L * q, 2 * L)]
                a, b = plsc.unpack(ab, format=plsc.PackFormat.INTERLEAVED)
                obuf[slot, e, pl.ds(2 * L * q, L)] = jnp.maximum(
                    a + ew_vec * w1bs[2 * q], 0.0)
                obuf[slot, e, pl.ds(2 * L * q + L, L)] = jnp.maximum(
                    b + ew_vec * w1bs[2 * q + 1], 0.0)

        plsc.parallel_loop(0, B, 1, unroll=EU)(_edge)

    def _chunk(ch, carry):
        pltpu.sync_copy(row4.at[s, ch], rbuf)
        pltpu.sync_copy(col4.at[s, ch], cbuf)
        pltpu.sync_copy(ew4.at[s, ch], ewbuf)
        _issue_g(0, 0)

        def _pair(jj2, pcarry):
            j0 = jj2 * 2
            j1 = j0 + 1
            _issue_g(j1, 1)

            @pl.when(c == 0)
            def _():
                _issue_c(j0)
                _issue_c(j1)

            _wait_g(j0, 0)

            @pl.when(jj2 > 0)
            def _():
                _wait_s(j0, 0)

            _compute(j0, 0)
            _issue_s(j0, 0)

            @pl.when(j1 + 1 < CHUNK)
            def _():
                _issue_g(j1 + 1, 0)

            _wait_g(j1, 1)

            @pl.when(jj2 > 0)
            def _():
                _wait_s(j1, 1)

            _compute(j1, 1)
            _issue_s(j1, 1)
            return pcarry

        lax.fori_loop(0, CHUNK // 2, _pair, 0)
        _wait_s(CHUNK - 2, 0)
        _wait_s(CHUNK - 1, 1)

        @pl.when(c == 0)
        def _():
            def _drain(jd, dcarry):
                _wait_c(jd)
                return dcarry
            lax.fori_loop(0, CHUNK, _drain, 0)

        return carry

    lax.fori_loop(0, NCHUNK, _chunk, 0)
    plsc.subcore_barrier()
    pltpu.sync_copy(acc_sh.at[pl.ds(s * RPT, RPT)],
                    acc_out.at[c, pl.ds(s * RPT, RPT)])

    @pl.when(c == 0)
    def _():
        pltpu.sync_copy(cnt_sh.at[pl.ds(s * RPT, RPT)],
                        cnt_out.at[pl.ds(s * RPT, RPT)])


def _edge_call(ytab0, ytab1, row4, col4, ew4, w1b2):
    mesh = plsc.VectorSubcoreMesh(core_axis_name="c", subcore_axis_name="s",
                                  num_cores=NC, num_subcores=NS)
    return pl.kernel(
        _edge_body,
        out_type=[
            jax.ShapeDtypeStruct((NC, N, H), _F32),
            jax.ShapeDtypeStruct((N, CW), _F32),
        ],
        mesh=mesh,
        scratch_types=[
            pltpu.VMEM((CHUNK, B), jnp.int32),
            pltpu.VMEM((CHUNK, B), jnp.int32),
            pltpu.VMEM((CHUNK, B), _F32),
            pltpu.VMEM((2, B, H), jnp.bfloat16),
            pltpu.VMEM((2, B, H), _F32),
            pltpu.VMEM((B, CW), _F32),
            pltpu.VMEM((H,), _F32),
            pltpu.VMEM_SHARED((N, H), _F32),
            pltpu.VMEM_SHARED((N, CW), _F32),
            pltpu.SemaphoreType.DMA,
            pltpu.SemaphoreType.DMA,
            pltpu.SemaphoreType.DMA,
            pltpu.SemaphoreType.DMA,
            pltpu.SemaphoreType.DMA,
        ],
        compiler_params=pltpu.CompilerParams(use_tc_tiling_on_sc=False,
                                             needs_layout_passes=False),
    )(ytab0, ytab1, row4, col4, ew4, w1b2)


def _e_body(x_ref, tgw_ref, gx_ref, a0_ref, a1_ref, cnt_ref,
            w2x_ref, w2n_ref, b2_ref, w2aT_ref, w2bT_ref, out_ref):
    x = x_ref[...]
    new_x = jnp.dot(tgw_ref[...], gx_ref[...], preferred_element_type=_F32)
    cnt = jnp.maximum(cnt_ref[:, :1], 1.0)
    a0 = a0_ref[0] / cnt
    a1 = a1_ref[0] / cnt
    o = lax.dot_general(x, w2x_ref[...], _CT11, preferred_element_type=_F32)
    o = o + lax.dot_general(new_x, w2n_ref[...], _CT11,
                            preferred_element_type=_F32)
    o = o + jnp.dot(a0, w2aT_ref[...], preferred_element_type=_F32)
    o = o + jnp.dot(a1, w2bT_ref[...], preferred_element_type=_F32)
    out_ref[...] = jnp.maximum(o + b2_ref[...], 0.0)


def _call_e(x, tgw, gx, acc, cnt, w2, b2, w2aT, w2bT):
    return pl.pallas_call(
        _e_body,
        grid=(NB,),
        in_specs=[
            pl.BlockSpec((BN, D), lambda i: (i, 0)),
            pl.BlockSpec((BN, G), lambda i: (i, 0)),
            pl.BlockSpec((G, D), lambda i: (0, 0)),
            pl.BlockSpec((1, BN, H), lambda i: (0, i, 0)),
            pl.BlockSpec((1, BN, H), lambda i: (1, i, 0)),
            pl.BlockSpec((BN, CW), lambda i: (i, 0)),
            pl.BlockSpec((D, D), lambda i: (0, 0)),
            pl.BlockSpec((D, D), lambda i: (0, 1)),
            pl.BlockSpec((1, D), lambda i: (0, 0)),
            pl.BlockSpec((H, D), lambda i: (0, 0)),
            pl.BlockSpec((H, D), lambda i: (0, 0)),
        ],
        out_specs=pl.BlockSpec((BN, D), lambda i: (i, 0)),
        out_shape=jax.ShapeDtypeStruct((N, D), _F32),
    )(x, tgw, gx, acc, acc, cnt, w2, w2, b2, w2aT, w2bT)


def kernel(input, tar_group_weights, enc_weights, group_edge_ids,
           group_edge_weights, edge_ids, edge_weights, emb_w, emb_b,
           g_fc1_w, g_fc1_b, g_fc2_w, g_fc2_b, glob_fc1_w, glob_fc1_b,
           glob_fc2_w, glob_fc2_b):
    f32 = _F32
    embb = emb_b.reshape(1, D)
    g1e = g_fc1_w[:, D:]
    g1b = g_fc1_b.reshape(1, D)
    g2b = g_fc2_b.reshape(1, D)
    w1b_col = glob_fc1_w[:, 2 * D]
    b1 = glob_fc1_b.reshape(1, D)
    w2aT = glob_fc2_w[:, 2 * D:2 * D + H].T[_PERM, :]
    w2bT = glob_fc2_w[:, 2 * D + H:].T[_PERM, :]
    b2 = glob_fc2_b.reshape(1, D)

    grow = group_edge_ids[0].astype(jnp.int32).reshape(EG, 1)
    gcol = group_edge_ids[1].astype(jnp.int32).reshape(EG, 1)

    row = edge_ids[0].astype(jnp.int32)
    col = edge_ids[1].astype(jnp.int32)
    row4 = row.reshape(NS, NCHUNK, CHUNK, B)
    col4 = col.reshape(NS, NCHUNK, CHUNK, B)
    ew4 = edge_weights.reshape(E,).astype(f32).reshape(NS, NCHUNK, CHUNK, B)
    w1b2 = w1b_col.reshape(NC, H)[:, _PERM]

    x, gx, _ = _call_a(input, emb_w, embb, tar_group_weights, grow, gcol,
                       group_edge_weights, g_fc1_w, g1e, g1b, g_fc2_w, g2b)
    ytab0, ytab1 = _call_c(x, tar_group_weights, gx, glob_fc1_w, b1)
    acc, cnt = _edge_call(ytab0, ytab1, row4, col4, ew4, w1b2)
    acc, cnt, _, _, _, _, _, _ = lax.optimization_barrier(
        (acc, cnt, ytab0, ytab1, row4, col4, ew4, w1b2))
    out = _call_e(x, tar_group_weights, gx, acc, cnt, glob_fc2_w, b2,
                  w2aT, w2bT)
    return out
